# Initial kernel scaffold; baseline (speedup 1.0000x reference)
import jax
import jax.numpy as jnp
from jax.experimental import pallas as pl


def kernel(positions, q, batch):
    raise NotImplementedError("write your pallas kernel here")


# speedup vs baseline: 1.3897x; 1.3897x over previous
import functools

import jax
import jax.numpy as jnp
from jax import lax
from jax.experimental import pallas as pl
from jax.experimental.pallas import tpu as pltpu
from jax.experimental.pallas import tpu_sc as plsc

N_POINTS = 1600000
NUM_SEG = 1024
ACC = NUM_SEG * 3

_info = plsc.get_sparse_core_info()
NUM_CORES = _info.num_cores
NUM_SUBCORES = _info.num_subcores
NW = NUM_CORES * NUM_SUBCORES
ROWS_PER_W = N_POINTS // NW
CHUNK = 10000
NCHUNK = ROWS_PER_W // CHUNK
VREGS = CHUNK // 16


def _sc_partials(posflat, q, batch):
  mesh = plsc.VectorSubcoreMesh(core_axis_name="c", subcore_axis_name="s")

  @functools.partial(
      pl.kernel,
      mesh=mesh,
      compiler_params=pltpu.CompilerParams(needs_layout_passes=False),
      out_type=[
          jax.ShapeDtypeStruct((NW, ACC), jnp.float32),
          jax.ShapeDtypeStruct((NW, ACC), jnp.float32),
          jax.ShapeDtypeStruct((NW, 16), jnp.float32),
      ],
      scratch_types=[
          pltpu.VMEM((3 * CHUNK,), jnp.float32),
          pltpu.VMEM((CHUNK,), jnp.float32),
          pltpu.VMEM((CHUNK,), jnp.int32),
          pltpu.VMEM((ACC,), jnp.float32),
          pltpu.VMEM((ACC,), jnp.float32),
          pltpu.VMEM((16,), jnp.float32),
      ],
  )
  def body(pos_hbm, q_hbm, b_hbm, out_qp, out_p, out_qs,
           posbuf, qbuf, bbuf, acc_qp, acc_p, acc_qs):
    wid = lax.axis_index("s") * NUM_CORES + lax.axis_index("c")
    row0 = wid * ROWS_PER_W

    zeros = jnp.zeros((16,), jnp.float32)

    def zero_body(j, _):
      acc_qp[pl.ds(j * 16, 16)] = zeros
      acc_p[pl.ds(j * 16, 16)] = zeros
      return 0

    lax.fori_loop(0, ACC // 16, zero_body, 0)
    acc_qs[...] = zeros

    iota16 = lax.iota(jnp.int32, 16)
    iota3 = iota16 * 3

    def chunk_body(c, _):
      r0 = row0 + c * CHUNK
      pltpu.sync_copy(pos_hbm.at[pl.ds(3 * r0, 3 * CHUNK)], posbuf)
      pltpu.sync_copy(q_hbm.at[pl.ds(r0, CHUNK)], qbuf)
      pltpu.sync_copy(b_hbm.at[pl.ds(r0, CHUNK)], bbuf)

      def vreg_body(k, _):
        k16 = k * 16
        qv = qbuf[pl.ds(k16, 16)]
        b = bbuf[pl.ds(k16, 16)]
        ix = iota3 + (k * 48)
        x = plsc.load_gather(posbuf, [ix])
        y = plsc.load_gather(posbuf, [ix + 1])
        z = plsc.load_gather(posbuf, [ix + 2])
        b3 = b * 3
        plsc.addupdate_scatter(acc_qp, [b3], qv * x)
        plsc.addupdate_scatter(acc_qp, [b3 + 1], qv * y)
        plsc.addupdate_scatter(acc_qp, [b3 + 2], qv * z)
        plsc.addupdate_scatter(acc_p, [b3], x)
        plsc.addupdate_scatter(acc_p, [b3 + 1], y)
        plsc.addupdate_scatter(acc_p, [b3 + 2], z)
        acc_qs[...] = acc_qs[...] + qv
        return 0

      lax.fori_loop(0, VREGS, vreg_body, 0)
      return 0

    lax.fori_loop(0, NCHUNK, chunk_body, 0)

    pltpu.sync_copy(acc_qp, out_qp.at[wid])
    pltpu.sync_copy(acc_p, out_p.at[wid])
    pltpu.sync_copy(acc_qs, out_qs.at[wid])

  return body(posflat, q, batch)


def _combine_body(qp_ref, p_ref, qs_ref, out_ref):
  m = jnp.sum(qs_ref[...]) * (1.0 / N_POINTS)
  out_ref[...] = (jnp.sum(qp_ref[...], axis=0, keepdims=True)
                  - m * jnp.sum(p_ref[...], axis=0, keepdims=True))


def kernel(positions, q, batch):
  posflat = positions.reshape(-1)
  qp_part, p_part, qs_part = _sc_partials(posflat, q, batch)
  out = pl.pallas_call(
      _combine_body,
      out_shape=jax.ShapeDtypeStruct((1, ACC), jnp.float32),
  )(qp_part, p_part, qs_part)
  return out.reshape(NUM_SEG, 3)


# speedup vs baseline: 17.2020x; 12.3785x over previous
import functools

import jax
import jax.numpy as jnp
from jax import lax
from jax.experimental import pallas as pl
from jax.experimental.pallas import tpu as pltpu
from jax.experimental.pallas import tpu_sc as plsc

N_POINTS = 1600000
NUM_SEG = 1024
ACC = NUM_SEG * 3

_info = plsc.get_sparse_core_info()
NUM_CORES = _info.num_cores
NUM_SUBCORES = _info.num_subcores
NW = NUM_CORES * NUM_SUBCORES
ROWS_PER_W = N_POINTS // NW
CHUNK = 10000
NCHUNK = ROWS_PER_W // CHUNK
VREGS = CHUNK // 16


def _sc_partials(x, y, z, q, batch):
  mesh = plsc.VectorSubcoreMesh(core_axis_name="c", subcore_axis_name="s")

  @functools.partial(
      pl.kernel,
      mesh=mesh,
      compiler_params=pltpu.CompilerParams(needs_layout_passes=False),
      out_type=[
          jax.ShapeDtypeStruct((NW, ACC), jnp.float32),
          jax.ShapeDtypeStruct((NW, ACC), jnp.float32),
          jax.ShapeDtypeStruct((NW, 16), jnp.float32),
      ],
      scratch_types=[
          pltpu.VMEM((CHUNK,), jnp.float32),
          pltpu.VMEM((CHUNK,), jnp.float32),
          pltpu.VMEM((CHUNK,), jnp.float32),
          pltpu.VMEM((CHUNK,), jnp.float32),
          pltpu.VMEM((CHUNK,), jnp.int32),
          pltpu.VMEM((ACC,), jnp.float32),
          pltpu.VMEM((ACC,), jnp.float32),
          pltpu.VMEM((16,), jnp.float32),
      ],
  )
  def body(x_hbm, y_hbm, z_hbm, q_hbm, b_hbm, out_qp, out_p, out_qs,
           xbuf, ybuf, zbuf, qbuf, bbuf, acc_qp, acc_p, acc_qs):
    wid = lax.axis_index("s") * NUM_CORES + lax.axis_index("c")
    row0 = wid * ROWS_PER_W

    zeros = jnp.zeros((16,), jnp.float32)

    def zero_body(j, _):
      acc_qp[pl.ds(j * 16, 16)] = zeros
      acc_p[pl.ds(j * 16, 16)] = zeros
      return 0

    lax.fori_loop(0, ACC // 16, zero_body, 0)
    acc_qs[...] = zeros

    def chunk_body(c, _):
      r0 = row0 + c * CHUNK
      pltpu.sync_copy(x_hbm.at[pl.ds(r0, CHUNK)], xbuf)
      pltpu.sync_copy(y_hbm.at[pl.ds(r0, CHUNK)], ybuf)
      pltpu.sync_copy(z_hbm.at[pl.ds(r0, CHUNK)], zbuf)
      pltpu.sync_copy(q_hbm.at[pl.ds(r0, CHUNK)], qbuf)
      pltpu.sync_copy(b_hbm.at[pl.ds(r0, CHUNK)], bbuf)

      def vreg_body(k, _):
        k16 = k * 16
        qv = qbuf[pl.ds(k16, 16)]
        b = bbuf[pl.ds(k16, 16)]
        xv = xbuf[pl.ds(k16, 16)]
        yv = ybuf[pl.ds(k16, 16)]
        zv = zbuf[pl.ds(k16, 16)]
        b3 = b * 3
        plsc.addupdate_scatter(acc_qp, [b3], qv * xv)
        plsc.addupdate_scatter(acc_qp, [b3 + 1], qv * yv)
        plsc.addupdate_scatter(acc_qp, [b3 + 2], qv * zv)
        plsc.addupdate_scatter(acc_p, [b3], xv)
        plsc.addupdate_scatter(acc_p, [b3 + 1], yv)
        plsc.addupdate_scatter(acc_p, [b3 + 2], zv)
        acc_qs[...] = acc_qs[...] + qv
        return 0

      lax.fori_loop(0, VREGS, vreg_body, 0)
      return 0

    lax.fori_loop(0, NCHUNK, chunk_body, 0)

    pltpu.sync_copy(acc_qp, out_qp.at[wid])
    pltpu.sync_copy(acc_p, out_p.at[wid])
    pltpu.sync_copy(acc_qs, out_qs.at[wid])

  return body(x, y, z, q, batch)


def _combine_body(qp_ref, p_ref, qs_ref, out_ref):
  m = jnp.sum(qs_ref[...]) * (1.0 / N_POINTS)
  out_ref[...] = (jnp.sum(qp_ref[...], axis=0, keepdims=True)
                  - m * jnp.sum(p_ref[...], axis=0, keepdims=True))


def kernel(positions, q, batch):
  x = positions[:, 0]
  y = positions[:, 1]
  z = positions[:, 2]
  qp_part, p_part, qs_part = _sc_partials(x, y, z, q, batch)
  out = pl.pallas_call(
      _combine_body,
      out_shape=jax.ShapeDtypeStruct((1, ACC), jnp.float32),
  )(qp_part, p_part, qs_part)
  return out.reshape(NUM_SEG, 3)


# speedup vs baseline: 17.7921x; 1.0343x over previous
import functools

import jax
import jax.numpy as jnp
from jax import lax
from jax.experimental import pallas as pl
from jax.experimental.pallas import tpu as pltpu
from jax.experimental.pallas import tpu_sc as plsc

N_POINTS = 1600000
NUM_SEG = 1024
ACC = NUM_SEG * 3

_info = plsc.get_sparse_core_info()
NUM_CORES = _info.num_cores
NUM_SUBCORES = _info.num_subcores
NW = NUM_CORES * NUM_SUBCORES
ROWS_PER_W = N_POINTS // NW
CHUNK = 10000
NCHUNK = ROWS_PER_W // CHUNK
VREGS = CHUNK // 16
UNROLL = 5


def _sc_partials(x, y, z, q, batch):
  mesh = plsc.VectorSubcoreMesh(core_axis_name="c", subcore_axis_name="s")

  @functools.partial(
      pl.kernel,
      mesh=mesh,
      compiler_params=pltpu.CompilerParams(needs_layout_passes=False),
      out_type=[
          jax.ShapeDtypeStruct((NW, ACC), jnp.float32),
          jax.ShapeDtypeStruct((NW, ACC), jnp.float32),
          jax.ShapeDtypeStruct((NW, 16), jnp.float32),
      ],
      scratch_types=[
          pltpu.VMEM((CHUNK,), jnp.float32),
          pltpu.VMEM((CHUNK,), jnp.float32),
          pltpu.VMEM((CHUNK,), jnp.float32),
          pltpu.VMEM((CHUNK,), jnp.float32),
          pltpu.VMEM((CHUNK,), jnp.int32),
          pltpu.VMEM((ACC,), jnp.float32),
          pltpu.VMEM((ACC,), jnp.float32),
          pltpu.VMEM((16,), jnp.float32),
      ],
  )
  def body(x_hbm, y_hbm, z_hbm, q_hbm, b_hbm, out_qp, out_p, out_qs,
           xbuf, ybuf, zbuf, qbuf, bbuf, acc_qp, acc_p, acc_qs):
    wid = lax.axis_index("s") * NUM_CORES + lax.axis_index("c")
    row0 = wid * ROWS_PER_W

    zeros = jnp.zeros((16,), jnp.float32)

    def zero_body(j, _):
      acc_qp[pl.ds(j * 16, 16)] = zeros
      acc_p[pl.ds(j * 16, 16)] = zeros
      return 0

    lax.fori_loop(0, ACC // 16, zero_body, 0)
    acc_qs[...] = zeros

    def chunk_body(c, _):
      r0 = row0 + c * CHUNK
      pltpu.sync_copy(x_hbm.at[pl.ds(r0, CHUNK)], xbuf)
      pltpu.sync_copy(y_hbm.at[pl.ds(r0, CHUNK)], ybuf)
      pltpu.sync_copy(z_hbm.at[pl.ds(r0, CHUNK)], zbuf)
      pltpu.sync_copy(q_hbm.at[pl.ds(r0, CHUNK)], qbuf)
      pltpu.sync_copy(b_hbm.at[pl.ds(r0, CHUNK)], bbuf)

      def vreg_body(k, _):
        qs = []
        for u in range(UNROLL):
          k16 = (k * UNROLL + u) * 16
          qv = qbuf[pl.ds(k16, 16)]
          b = bbuf[pl.ds(k16, 16)]
          xv = xbuf[pl.ds(k16, 16)]
          yv = ybuf[pl.ds(k16, 16)]
          zv = zbuf[pl.ds(k16, 16)]
          b3 = b * 3
          plsc.addupdate_scatter(acc_qp, [b3], qv * xv)
          plsc.addupdate_scatter(acc_qp, [b3 + 1], qv * yv)
          plsc.addupdate_scatter(acc_qp, [b3 + 2], qv * zv)
          plsc.addupdate_scatter(acc_p, [b3], xv)
          plsc.addupdate_scatter(acc_p, [b3 + 1], yv)
          plsc.addupdate_scatter(acc_p, [b3 + 2], zv)
          qs.append(qv)
        while len(qs) > 1:
          qs = [a + b for a, b in zip(qs[::2], qs[1::2])] + (
              [qs[-1]] if len(qs) % 2 else [])
        acc_qs[...] = acc_qs[...] + qs[0]
        return 0

      lax.fori_loop(0, VREGS // UNROLL, vreg_body, 0)
      return 0

    lax.fori_loop(0, NCHUNK, chunk_body, 0)

    pltpu.sync_copy(acc_qp, out_qp.at[wid])
    pltpu.sync_copy(acc_p, out_p.at[wid])
    pltpu.sync_copy(acc_qs, out_qs.at[wid])

  return body(x, y, z, q, batch)


def _combine_body(qp_ref, p_ref, qs_ref, out_ref):
  m = jnp.sum(qs_ref[...]) * (1.0 / N_POINTS)
  out_ref[...] = (jnp.sum(qp_ref[...], axis=0, keepdims=True)
                  - m * jnp.sum(p_ref[...], axis=0, keepdims=True))


def kernel(positions, q, batch):
  x = positions[:, 0]
  y = positions[:, 1]
  z = positions[:, 2]
  qp_part, p_part, qs_part = _sc_partials(x, y, z, q, batch)
  out = pl.pallas_call(
      _combine_body,
      out_shape=jax.ShapeDtypeStruct((1, ACC), jnp.float32),
  )(qp_part, p_part, qs_part)
  return out.reshape(NUM_SEG, 3)


# speedup vs baseline: 45.1434x; 2.5373x over previous
import functools

import jax
import jax.numpy as jnp
from jax import lax
from jax.experimental import pallas as pl
from jax.experimental.pallas import tpu as pltpu
from jax.experimental.pallas import tpu_sc as plsc

N_POINTS = 1600000
NUM_SEG = 1024
ACC = NUM_SEG * 3

_info = plsc.get_sparse_core_info()
NUM_CORES = _info.num_cores
NUM_SUBCORES = _info.num_subcores
NW = NUM_CORES * NUM_SUBCORES
ROWS_PER_W = N_POINTS // NW
CHUNK = 10000
NCHUNK = ROWS_PER_W // CHUNK
VREGS = CHUNK // 16
U = 5
BLK = U * 16
NBLK = VREGS // U


def _tree_sum(vs):
  while len(vs) > 1:
    vs = [a + b for a, b in zip(vs[::2], vs[1::2])] + (
        [vs[-1]] if len(vs) % 2 else [])
  return vs[0]


def _sc_partials(x, y, z, q, batch):
  mesh = plsc.VectorSubcoreMesh(core_axis_name="c", subcore_axis_name="s")

  @functools.partial(
      pl.kernel,
      mesh=mesh,
      compiler_params=pltpu.CompilerParams(needs_layout_passes=False),
      out_type=[
          jax.ShapeDtypeStruct((NW, ACC), jnp.float32),
          jax.ShapeDtypeStruct((NW, ACC), jnp.float32),
          jax.ShapeDtypeStruct((NW, 16), jnp.float32),
      ],
      scratch_types=[
          pltpu.VMEM((CHUNK,), jnp.float32),
          pltpu.VMEM((CHUNK,), jnp.float32),
          pltpu.VMEM((CHUNK,), jnp.float32),
          pltpu.VMEM((CHUNK,), jnp.float32),
          pltpu.VMEM((CHUNK,), jnp.int32),
          pltpu.VMEM((ACC,), jnp.float32),
          pltpu.VMEM((ACC,), jnp.float32),
          pltpu.VMEM((16,), jnp.float32),
      ],
  )
  def body(x_hbm, y_hbm, z_hbm, q_hbm, b_hbm, out_qp, out_p, out_qs,
           xbuf, ybuf, zbuf, qbuf, bbuf, acc_qp, acc_p, qs_buf):
    wid = lax.axis_index("s") * NUM_CORES + lax.axis_index("c")
    row0 = wid * ROWS_PER_W

    zeros = jnp.zeros((16,), jnp.float32)
    zeros_i = jnp.zeros((16,), jnp.int32)

    def zero_body(j, _):
      acc_qp[pl.ds(j * 16, 16)] = zeros
      acc_p[pl.ds(j * 16, 16)] = zeros
      return 0

    lax.fori_loop(0, ACC // 16, zero_body, 0)

    def flush(cur, vqx, vqy, vqz, vpx, vpy, vpz):
      i0 = zeros_i + jnp.maximum(cur, 0) * 3
      plsc.addupdate_scatter(acc_qp, [i0], vqx)
      plsc.addupdate_scatter(acc_qp, [i0 + 1], vqy)
      plsc.addupdate_scatter(acc_qp, [i0 + 2], vqz)
      plsc.addupdate_scatter(acc_p, [i0], vpx)
      plsc.addupdate_scatter(acc_p, [i0 + 1], vpy)
      plsc.addupdate_scatter(acc_p, [i0 + 2], vpz)

    def chunk_body(c, carry):
      r0 = row0 + c * CHUNK
      pltpu.sync_copy(x_hbm.at[pl.ds(r0, CHUNK)], xbuf)
      pltpu.sync_copy(y_hbm.at[pl.ds(r0, CHUNK)], ybuf)
      pltpu.sync_copy(z_hbm.at[pl.ds(r0, CHUNK)], zbuf)
      pltpu.sync_copy(q_hbm.at[pl.ds(r0, CHUNK)], qbuf)
      pltpu.sync_copy(b_hbm.at[pl.ds(r0, CHUNK)], bbuf)

      def blk_body(k, carry):
        cur, vqx, vqy, vqz, vpx, vpy, vpz, vqs = carry
        k0 = k * BLK
        b_first = bbuf[pl.ds(k0, 16)][0]
        b_last = bbuf[pl.ds(k0 + BLK - 16, 16)][15]
        is_fast = jnp.logical_and(b_first == cur, b_last == b_first)

        def fast_fn(carry):
          cur, vqx, vqy, vqz, vpx, vpy, vpz, vqs = carry
          xs, ys, zs, qs, qxs, qys, qzs = [], [], [], [], [], [], []
          for u in range(U):
            o = k0 + u * 16
            xv = xbuf[pl.ds(o, 16)]
            yv = ybuf[pl.ds(o, 16)]
            zv = zbuf[pl.ds(o, 16)]
            qv = qbuf[pl.ds(o, 16)]
            xs.append(xv); ys.append(yv); zs.append(zv); qs.append(qv)
            qxs.append(qv * xv); qys.append(qv * yv); qzs.append(qv * zv)
          return (cur,
                  vqx + _tree_sum(qxs), vqy + _tree_sum(qys),
                  vqz + _tree_sum(qzs),
                  vpx + _tree_sum(xs), vpy + _tree_sum(ys),
                  vpz + _tree_sum(zs), vqs + _tree_sum(qs))

        def slow_fn(carry):
          cur, vqx, vqy, vqz, vpx, vpy, vpz, vqs = carry
          flush(cur, vqx, vqy, vqz, vpx, vpy, vpz)
          qs = []
          for u in range(U):
            o = k0 + u * 16
            xv = xbuf[pl.ds(o, 16)]
            yv = ybuf[pl.ds(o, 16)]
            zv = zbuf[pl.ds(o, 16)]
            qv = qbuf[pl.ds(o, 16)]
            b3 = bbuf[pl.ds(o, 16)] * 3
            plsc.addupdate_scatter(acc_qp, [b3], qv * xv)
            plsc.addupdate_scatter(acc_qp, [b3 + 1], qv * yv)
            plsc.addupdate_scatter(acc_qp, [b3 + 2], qv * zv)
            plsc.addupdate_scatter(acc_p, [b3], xv)
            plsc.addupdate_scatter(acc_p, [b3 + 1], yv)
            plsc.addupdate_scatter(acc_p, [b3 + 2], zv)
            qs.append(qv)
          return (b_last, zeros, zeros, zeros, zeros, zeros, zeros,
                  vqs + _tree_sum(qs))

        return lax.cond(is_fast, fast_fn, slow_fn, carry)

      return lax.fori_loop(0, NBLK, blk_body, carry)

    carry0 = (jnp.int32(-1), zeros, zeros, zeros, zeros, zeros, zeros, zeros)
    cur, vqx, vqy, vqz, vpx, vpy, vpz, vqs = lax.fori_loop(
        0, NCHUNK, chunk_body, carry0)
    flush(cur, vqx, vqy, vqz, vpx, vpy, vpz)
    qs_buf[...] = vqs

    pltpu.sync_copy(acc_qp, out_qp.at[wid])
    pltpu.sync_copy(acc_p, out_p.at[wid])
    pltpu.sync_copy(qs_buf, out_qs.at[wid])

  return body(x, y, z, q, batch)


def _combine_body(qp_ref, p_ref, qs_ref, out_ref):
  m = jnp.sum(qs_ref[...]) * (1.0 / N_POINTS)
  out_ref[...] = (jnp.sum(qp_ref[...], axis=0, keepdims=True)
                  - m * jnp.sum(p_ref[...], axis=0, keepdims=True))


def kernel(positions, q, batch):
  x = positions[:, 0]
  y = positions[:, 1]
  z = positions[:, 2]
  qp_part, p_part, qs_part = _sc_partials(x, y, z, q, batch)
  out = pl.pallas_call(
      _combine_body,
      out_shape=jax.ShapeDtypeStruct((1, ACC), jnp.float32),
  )(qp_part, p_part, qs_part)
  return out.reshape(NUM_SEG, 3)
